# baseline (speedup 1.0000x reference)
import functools

import jax
import jax.numpy as jnp
from jax import lax
from jax.experimental import pallas as pl
from jax.experimental.pallas import tpu as pltpu
from jax.experimental.pallas import tpu_sc as plsc

N = 10000
E = 160000
NUM_ELEM = 100
EMBED = 256
EMB32 = EMBED // 2
HID = 512
NG = 50
NGP = 64
NH = 8
NM = 1
RBF_R = 12.0

NC = 2
NS = 16
NW = NC * NS
L = 16
EPW = E // NW
PAD = 8
CH = 200
NCHUNK = EPW // CH

RPT = 624
RCH = 24
NRCH = RPT // RCH
TAILR = N - NS * RPT

TE = 3200
GRID = E // TE

_STEP = RBF_R / (NG - 1)
_COEFF = -0.5 / _STEP ** 2

_MESH = plsc.VectorSubcoreMesh(core_axis_name="c", subcore_axis_name="s")


def _sc_fused_body(anum_hbm, e0_hbm, e1_hbm, src_hbm, table_hbm,
                   out_hbm, tpack_hbm,
                   anum_v, idx_v, s0_v, s1_v, rows0_v, rows1_v,
                   pin_v, pout_v, gsem0, gsem1, osem0, osem1):
    core = lax.axis_index("c")
    sub = lax.axis_index("s")
    wid = sub * NC + core
    base = wid * EPW
    idx_v[pl.ds(EPW + PAD - L, L)] = jnp.zeros((L,), jnp.int32)
    pltpu.sync_copy(src_hbm.at[pl.ds(base, EPW)], idx_v.at[pl.ds(0, EPW)])
    c0 = pltpu.async_copy(e0_hbm.at[idx_v], s0_v, gsem0)
    c1 = pltpu.async_copy(e1_hbm.at[idx_v], s1_v, gsem1)
    pltpu.sync_copy(anum_hbm, anum_v)

    def row_step(r, carry):
        rv = jnp.zeros((L,), jnp.int32) + r
        for j in range(EMBED // 32):
            ii = j * 32 + 2 * lax.iota(jnp.int32, 16)
            even = plsc.load_gather(pin_v, [rv, ii])
            odd = plsc.load_gather(pin_v, [rv, ii + 1])
            packed = plsc.pack(even, odd,
                               format=plsc.PackFormat.INTERLEAVED)
            pout_v[r, pl.ds(j * L, L)] = plsc.bitcast(packed, jnp.int32)
        return carry

    for c in range(NRCH):
        rbase = sub * RPT + c * RCH
        pltpu.sync_copy(table_hbm.at[pl.ds(rbase, RCH)], pin_v)
        lax.fori_loop(0, RCH, row_step, 0)
        pltpu.sync_copy(pout_v, tpack_hbm.at[pl.ds(core * N + rbase, RCH)])

    @pl.when(sub < 2)
    def _():
        tbase = NS * RPT + sub * (TAILR // 2)
        pltpu.sync_copy(table_hbm.at[pl.ds(tbase, TAILR // 2)],
                        pin_v.at[pl.ds(0, TAILR // 2)])
        lax.fori_loop(0, TAILR // 2, row_step, 0)
        pltpu.sync_copy(pout_v.at[pl.ds(0, TAILR // 2)],
                        tpack_hbm.at[pl.ds(core * N + tbase, TAILR // 2)])

    c0.wait()
    c1.wait()
    row0 = core * N

    def pair_step(i, carry):
        sl = pl.ds(i * L, L)
        s0 = jnp.clip(s0_v[sl], 0, N - 1)
        s1 = jnp.clip(s1_v[sl], 0, N - 1)
        a0 = plsc.load_gather(anum_v, [s0])
        a1 = plsc.load_gather(anum_v, [s1])
        idx_v[sl] = a0 + NUM_ELEM * a1 + row0
        return carry

    lax.fori_loop(0, (EPW + PAD) // L, pair_step, 0)

    plsc.subcore_barrier()

    bufs = (rows0_v, rows1_v)
    gsems = (gsem0, gsem1)
    osems = (osem0, osem1)

    def fire(c):
        return pltpu.async_copy(
            tpack_hbm.at[idx_v.at[pl.ds(c * CH, CH)]], bufs[c & 1],
            gsems[c & 1])

    g = [None] * NCHUNK
    o = [None] * NCHUNK
    g[0] = fire(0)
    for c in range(NCHUNK):
        p = c & 1
        if c + 1 < NCHUNK:
            if c >= 1:
                o[c - 1].wait()
            g[c + 1] = fire(c + 1)
        g[c].wait()
        o[c] = pltpu.async_copy(
            bufs[p], out_hbm.at[pl.ds(base + c * CH, CH)], osems[p])
    o[NCHUNK - 2].wait()
    o[NCHUNK - 1].wait()


_sc_fused = functools.partial(
    pl.kernel,
    out_type=(
        jax.ShapeDtypeStruct((E, EMB32), jnp.int32),
        jax.ShapeDtypeStruct((NC * N, EMB32), jnp.int32),
    ),
    mesh=_MESH,
    compiler_params=pltpu.CompilerParams(needs_layout_passes=False),
    scratch_types=[
        pltpu.VMEM((N,), jnp.int32),
        pltpu.VMEM((EPW + PAD,), jnp.int32),
        pltpu.VMEM((EPW + PAD,), jnp.int32),
        pltpu.VMEM((EPW + PAD,), jnp.int32),
        pltpu.VMEM((CH, EMB32), jnp.int32),
        pltpu.VMEM((CH, EMB32), jnp.int32),
        pltpu.VMEM((RCH, EMBED), jnp.float32),
        pltpu.VMEM((RCH, EMB32), jnp.int32),
        pltpu.SemaphoreType.DMA,
        pltpu.SemaphoreType.DMA,
        pltpu.SemaphoreType.DMA,
        pltpu.SemaphoreType.DMA,
    ],
)(_sc_fused_body)


def _tc_body(dist_ref, emb_ref, wie_ref, wrg_ref, bi_ref, wo_ref,
             bo_ref, out_ref):
    d = dist_ref[...]
    col = lax.broadcasted_iota(jnp.int32, (TE, NGP), 1).astype(jnp.float32)
    diff = d - col * _STEP
    rbf = jnp.exp(_COEFF * diff * diff)
    rbf_b = rbf.astype(jnp.bfloat16)
    x = emb_ref[...]
    even_b = lax.bitcast_convert_type(
        lax.shift_left(x, 16), jnp.float32).astype(jnp.bfloat16)
    odd_b = lax.bitcast_convert_type(
        lax.bitwise_and(x, jnp.int32(-65536)), jnp.float32).astype(jnp.bfloat16)
    eo = jnp.concatenate([even_b, odd_b], axis=1)
    dn = (((1,), (1,)), ((), ()))
    acc = lax.dot_general(eo, wie_ref[...], dn,
                          preferred_element_type=jnp.float32)
    rg = lax.dot_general(rbf_b, wrg_ref[...], dn,
                         preferred_element_type=jnp.float32)
    acc = acc + rg[:, :HID] + bi_ref[...]
    h = acc * lax.logistic(acc) * rg[:, HID:]
    o = lax.dot_general(wo_ref[...], h.astype(jnp.bfloat16), dn,
                        preferred_element_type=jnp.float32)
    out_ref[...] = o + bo_ref[...]


_tc_mlp = pl.pallas_call(
    _tc_body,
    grid=(GRID,),
    in_specs=[
        pl.BlockSpec((TE, 1), lambda i: (i, 0)),
        pl.BlockSpec((TE, EMB32), lambda i: (i, 0)),
        pl.BlockSpec((HID, EMBED), lambda i: (0, 0)),
        pl.BlockSpec((2 * HID, NGP), lambda i: (0, 0)),
        pl.BlockSpec((1, HID), lambda i: (0, 0)),
        pl.BlockSpec((NH, HID), lambda i: (0, 0)),
        pl.BlockSpec((NH, 1), lambda i: (0, 0)),
    ],
    out_specs=pl.BlockSpec((NH, TE), lambda i: (0, i)),
    out_shape=jax.ShapeDtypeStruct((NH, E), jnp.float32),
)


def kernel(anum, edge_index, edge_to_src, dist, table, Wg, Wi, bi, Wo, bo):
    anum = anum.astype(jnp.int32)
    e0 = edge_index[0].astype(jnp.int32)
    e1 = edge_index[1].astype(jnp.int32)
    src = edge_to_src.astype(jnp.int32)
    emb_p, _ = _sc_fused(anum, e0, e1, src, table)
    wie_cat = jnp.concatenate(
        [Wi[:, 0:EMBED:2], Wi[:, 1:EMBED:2]], axis=1).astype(jnp.bfloat16)
    wrg_cat = jnp.concatenate(
        [jnp.pad(Wi[:, EMBED:], ((0, 0), (0, NGP - NG))),
         jnp.pad(Wg, ((0, 0), (0, NGP - NG)))], axis=0).astype(jnp.bfloat16)
    out2d = _tc_mlp(dist.reshape(E, 1), emb_p, wie_cat, wrg_cat,
                    bi.reshape(1, HID), Wo.astype(jnp.bfloat16),
                    bo.reshape(NH, 1))
    return out2d.reshape(NM, NH, E)


# speedup vs baseline: 1.0270x; 1.0270x over previous
import functools

import jax
import jax.numpy as jnp
from jax import lax
from jax.experimental import pallas as pl
from jax.experimental.pallas import tpu as pltpu
from jax.experimental.pallas import tpu_sc as plsc

N = 10000
E = 160000
NUM_ELEM = 100
EMBED = 256
EMB32 = EMBED // 2
HID = 512
NG = 50
NGP = 64
NH = 8
NM = 1
RBF_R = 12.0

NC = 2
NS = 16
NW = NC * NS
L = 16
NSLICE = 1
ES = E // NSLICE
EPW = ES // NW
PAD = 8
CH = 200
NCHUNK = EPW // CH

NWP = 25
RPW = N // NWP
RCH = 40
NRCH = RPW // RCH

TE = 3200
GRIDS = ES // TE

_STEP = RBF_R / (NG - 1)
_COEFF = -0.5 / _STEP ** 2

_MESH = plsc.VectorSubcoreMesh(core_axis_name="c", subcore_axis_name="s")


def _sc_pack_body(table_hbm, out_hbm, in_v, out_v):
    wid = lax.axis_index("s") * NC + lax.axis_index("c")

    @pl.when(wid < NWP)
    def _():
        for c in range(NRCH):
            rbase = wid * RPW + c * RCH
            pltpu.sync_copy(table_hbm.at[pl.ds(rbase, RCH)], in_v)

            def row_step(r, carry):
                rv = jnp.zeros((L,), jnp.int32) + r
                for j in range(EMBED // 32):
                    ii = j * 32 + 2 * lax.iota(jnp.int32, 16)
                    even = plsc.load_gather(in_v, [rv, ii])
                    odd = plsc.load_gather(in_v, [rv, ii + 1])
                    packed = plsc.pack(even, odd,
                                       format=plsc.PackFormat.INTERLEAVED)
                    out_v[r, pl.ds(j * L, L)] = plsc.bitcast(packed, jnp.int32)
                return carry

            lax.fori_loop(0, RCH, row_step, 0)
            pltpu.sync_copy(out_v, out_hbm.at[pl.ds(rbase, RCH)])


_sc_pack = functools.partial(
    pl.kernel,
    out_type=jax.ShapeDtypeStruct((N, EMB32), jnp.int32),
    mesh=_MESH,
    compiler_params=pltpu.CompilerParams(needs_layout_passes=False),
    scratch_types=[
        pltpu.VMEM((RCH, EMBED), jnp.float32),
        pltpu.VMEM((RCH, EMB32), jnp.int32),
    ],
)(_sc_pack_body)


def _make_sc_gather(slice_idx):
    off = slice_idx * ES

    def body(anum_hbm, e0_hbm, e1_hbm, src_hbm, table_hbm, out_hbm,
             anum_v, idx_v, s0_v, s1_v, rows0_v, rows1_v,
             gsem0, gsem1, osem0, osem1):
        wid = lax.axis_index("s") * NC + lax.axis_index("c")
        base = off + wid * EPW
        pltpu.sync_copy(anum_hbm, anum_v)
        idx_v[pl.ds(EPW + PAD - L, L)] = jnp.zeros((L,), jnp.int32)
        pltpu.sync_copy(src_hbm.at[pl.ds(base, EPW)], idx_v.at[pl.ds(0, EPW)])
        c0 = pltpu.async_copy(e0_hbm.at[idx_v], s0_v, gsem0)
        c1 = pltpu.async_copy(e1_hbm.at[idx_v], s1_v, gsem1)
        c0.wait()
        c1.wait()

        def pair_step(i, carry):
            sl = pl.ds(i * L, L)
            s0 = jnp.clip(s0_v[sl], 0, N - 1)
            s1 = jnp.clip(s1_v[sl], 0, N - 1)
            a0 = plsc.load_gather(anum_v, [s0])
            a1 = plsc.load_gather(anum_v, [s1])
            idx_v[sl] = a0 + NUM_ELEM * a1
            return carry

        lax.fori_loop(0, (EPW + PAD) // L, pair_step, 0)

        bufs = (rows0_v, rows1_v)
        gsems = (gsem0, gsem1)
        osems = (osem0, osem1)
        wbase = wid * EPW

        def fire(c):
            return pltpu.async_copy(
                table_hbm.at[idx_v.at[pl.ds(c * CH, CH)]], bufs[c & 1],
                gsems[c & 1])

        g = [None] * NCHUNK
        o = [None] * NCHUNK
        g[0] = fire(0)
        for c in range(NCHUNK):
            p = c & 1
            if c + 1 < NCHUNK:
                if c >= 1:
                    o[c - 1].wait()
                g[c + 1] = fire(c + 1)
            g[c].wait()
            o[c] = pltpu.async_copy(
                bufs[p], out_hbm.at[pl.ds(wbase + c * CH, CH)], osems[p])
        o[NCHUNK - 2].wait()
        o[NCHUNK - 1].wait()

    return functools.partial(
        pl.kernel,
        out_type=jax.ShapeDtypeStruct((ES, EMB32), jnp.int32),
        mesh=_MESH,
        compiler_params=pltpu.CompilerParams(needs_layout_passes=False),
        scratch_types=[
            pltpu.VMEM((N,), jnp.int32),
            pltpu.VMEM((EPW + PAD,), jnp.int32),
            pltpu.VMEM((EPW + PAD,), jnp.int32),
            pltpu.VMEM((EPW + PAD,), jnp.int32),
            pltpu.VMEM((CH, EMB32), jnp.int32),
            pltpu.VMEM((CH, EMB32), jnp.int32),
            pltpu.SemaphoreType.DMA,
            pltpu.SemaphoreType.DMA,
            pltpu.SemaphoreType.DMA,
            pltpu.SemaphoreType.DMA,
        ],
    )(body)


_sc_gathers = [_make_sc_gather(k) for k in range(NSLICE)]


def _tc_body(dist_ref, emb_ref, wie_ref, wrg_ref, bi_ref, wo_ref,
             bo_ref, out_ref):
    d = dist_ref[...]
    col = lax.broadcasted_iota(jnp.int32, (TE, NGP), 1).astype(jnp.float32)
    diff = d - col * _STEP
    rbf = jnp.exp(_COEFF * diff * diff)
    rbf_b = rbf.astype(jnp.bfloat16)
    x = emb_ref[...]
    even_b = lax.bitcast_convert_type(
        lax.shift_left(x, 16), jnp.float32).astype(jnp.bfloat16)
    odd_b = lax.bitcast_convert_type(
        lax.bitwise_and(x, jnp.int32(-65536)), jnp.float32).astype(jnp.bfloat16)
    eo = jnp.concatenate([even_b, odd_b], axis=1)
    dn = (((1,), (1,)), ((), ()))
    acc = lax.dot_general(eo, wie_ref[...], dn,
                          preferred_element_type=jnp.float32)
    rg = lax.dot_general(rbf_b, wrg_ref[...], dn,
                         preferred_element_type=jnp.float32)
    acc = acc + rg[:, :HID] + bi_ref[...]
    h = acc * lax.logistic(acc) * rg[:, HID:]
    o = lax.dot_general(wo_ref[...], h.astype(jnp.bfloat16), dn,
                        preferred_element_type=jnp.float32)
    out_ref[...] = o + bo_ref[...]


def _make_tc_mlp(slice_idx):
    roff = slice_idx * GRIDS
    return pl.pallas_call(
        _tc_body,
        grid=(GRIDS,),
        in_specs=[
            pl.BlockSpec((TE, 1), lambda i: (roff + i, 0)),
            pl.BlockSpec((TE, EMB32), lambda i: (i, 0)),
            pl.BlockSpec((HID, EMBED), lambda i: (0, 0)),
            pl.BlockSpec((2 * HID, NGP), lambda i: (0, 0)),
            pl.BlockSpec((1, HID), lambda i: (0, 0)),
            pl.BlockSpec((NH, HID), lambda i: (0, 0)),
            pl.BlockSpec((NH, 1), lambda i: (0, 0)),
        ],
        out_specs=pl.BlockSpec((NH, TE), lambda i: (0, i)),
        out_shape=jax.ShapeDtypeStruct((NH, ES), jnp.float32),
    )


_tc_mlps = [_make_tc_mlp(k) for k in range(NSLICE)]


def kernel(anum, edge_index, edge_to_src, dist, table, Wg, Wi, bi, Wo, bo):
    anum = anum.astype(jnp.int32)
    e0 = edge_index[0].astype(jnp.int32)
    e1 = edge_index[1].astype(jnp.int32)
    src = edge_to_src.astype(jnp.int32)
    table_p = _sc_pack(table)
    wie_cat = jnp.concatenate(
        [Wi[:, 0:EMBED:2], Wi[:, 1:EMBED:2]], axis=1).astype(jnp.bfloat16)
    wrg_cat = jnp.concatenate(
        [jnp.pad(Wi[:, EMBED:], ((0, 0), (0, NGP - NG))),
         jnp.pad(Wg, ((0, 0), (0, NGP - NG)))], axis=0).astype(jnp.bfloat16)
    dist2 = dist.reshape(E, 1)
    bi2 = bi.reshape(1, HID)
    bo2 = bo.reshape(NH, 1)
    wo_b = Wo.astype(jnp.bfloat16)
    outs = []
    for k in range(NSLICE):
        emb_k = _sc_gathers[k](anum, e0, e1, src, table_p)
        outs.append(_tc_mlps[k](dist2, emb_k, wie_cat, wrg_cat,
                                bi2, wo_b, bo2))
    return jnp.concatenate(outs, axis=1).reshape(NM, NH, E)
